# Initial kernel scaffold; baseline (speedup 1.0000x reference)
import jax
import jax.numpy as jnp
from jax.experimental import pallas as pl


def kernel(xyz, edges, tables):
    raise NotImplementedError("write your pallas kernel here")


# speedup vs baseline: 116.9918x; 116.9918x over previous
import functools

import jax
import jax.numpy as jnp
from jax import lax
from jax.experimental import pallas as pl
from jax.experimental.pallas import tpu as pltpu
from jax.experimental.pallas import tpu_sc as plsc

N_NODES = 10000
K_NBR = 16
HEADS = 2
NENC = 6
TSIZE = 4096
FEAT = 2
LEVELS = 8
_B = (1.0e7) ** (1.0 / (LEVELS - 1))
_RES = [float(_B**l) for l in range(LEVELS)]
_P1 = 2654435761
_P2 = 805459861

NW = 32
NPAD = 10240
CPT = NPAD // NW
NCHUNK = 32
CHUNKS = CPT // NCHUNK
ROW = K_NBR * LEVELS * FEAT
TWORDS = LEVELS * TSIZE * FEAT


def _body(xyz_h, edges_h, tab_h,
          o0, o1, o2, o3, o4, o5,
          xyz_v, edges_v, relx, rely, relz, tab_v, obuf):
    info = plsc.get_sparse_core_info()
    nc = info.num_cores
    wid = lax.axis_index("s") * nc + lax.axis_index("c")
    n0 = wid * CPT

    iota16 = lax.iota(jnp.int32, 16)
    colbase = iota16 * 16

    pltpu.sync_copy(xyz_h, xyz_v)
    pltpu.sync_copy(edges_h.at[pl.ds(n0 * K_NBR, CPT * K_NBR)], edges_v)

    def rel_body(j, carry):
        erow3 = edges_v[pl.ds(j * 16, 16)] * 3
        ex = plsc.load_gather(xyz_v, [erow3])
        ey = plsc.load_gather(xyz_v, [erow3 + 1])
        ez = plsc.load_gather(xyz_v, [erow3 + 2])
        own3 = jnp.full((16,), (n0 + j) * 3, jnp.int32)
        ox = plsc.load_gather(xyz_v, [own3])
        oy = plsc.load_gather(xyz_v, [own3 + 1])
        oz = plsc.load_gather(xyz_v, [own3 + 2])
        relx[pl.ds(j * 16, 16)] = ox - ex
        rely[pl.ds(j * 16, 16)] = oy - ey
        relz[pl.ds(j * 16, 16)] = oz - ez
        return carry

    lax.fori_loop(0, CPT, rel_body, 0)

    outs = (o0, o1, o2, o3, o4, o5)

    for e in range(NENC):
        pltpu.sync_copy(tab_h.at[pl.ds(e * TWORDS, TWORDS)], tab_v)
        out_ref = outs[e]

        def node_body(jj, carry, c):
            j = c * NCHUNK + jj
            rx = relx[pl.ds(j * 16, 16)]
            ry = rely[pl.ds(j * 16, 16)]
            rz = relz[pl.ds(j * 16, 16)]
            rowbase = jnp.full((16,), jj * ROW, jnp.int32) + colbase
            for l in range(LEVELS):
                res = jnp.float32(_RES[l])
                px = rx * res
                py = ry * res
                pz = rz * res
                ix = px.astype(jnp.int32)
                iy = py.astype(jnp.int32)
                iz = pz.astype(jnp.int32)
                fx = px - ix.astype(jnp.float32)
                fy = py - iy.astype(jnp.float32)
                fz = pz - iz.astype(jnp.float32)
                nx = fx < 0.0
                ny = fy < 0.0
                nz = fz < 0.0
                ix = jnp.where(nx, ix - 1, ix)
                iy = jnp.where(ny, iy - 1, iy)
                iz = jnp.where(nz, iz - 1, iz)
                fx = jnp.where(nx, fx + 1.0, fx)
                fy = jnp.where(ny, fy + 1.0, fy)
                fz = jnp.where(nz, fz + 1.0, fz)
                a0 = ix.astype(jnp.uint32)
                a1 = a0 + jnp.uint32(1)
                b0 = iy.astype(jnp.uint32) * jnp.uint32(_P1)
                b1 = b0 + jnp.uint32(_P1)
                c0 = iz.astype(jnp.uint32) * jnp.uint32(_P2)
                c1 = c0 + jnp.uint32(_P2)
                bc = ((b0 ^ c0, b1 ^ c0), (b0 ^ c1, b1 ^ c1))
                gx = 1.0 - fx
                gy = 1.0 - fy
                gz = 1.0 - fz
                wyz = ((gy * gz, fy * gz), (gy * fz, fy * fz))
                lbase = jnp.uint32(l * TSIZE * FEAT)
                acc0 = None
                acc1 = None
                for cz in range(2):
                    for cy in range(2):
                        for cx in range(2):
                            hsh = ((a1 if cx else a0) ^ bc[cz][cy]) & jnp.uint32(TSIZE - 1)
                            idx0 = (hsh * jnp.uint32(FEAT) + lbase).astype(jnp.int32)
                            f0 = plsc.load_gather(tab_v, [idx0])
                            f1 = plsc.load_gather(tab_v, [idx0 + 1])
                            w = (fx if cx else gx) * wyz[cz][cy]
                            if acc0 is None:
                                acc0 = w * f0
                                acc1 = w * f1
                            else:
                                acc0 = acc0 + w * f0
                                acc1 = acc1 + w * f1
                col0 = rowbase + (2 * l)
                plsc.store_scatter(obuf, [col0], acc0)
                plsc.store_scatter(obuf, [col0 + 1], acc1)
            return carry

        def chunk_body(c, carry):
            lax.fori_loop(0, NCHUNK, lambda jj, cy, c=c: node_body(jj, cy, c), 0)
            pltpu.sync_copy(obuf, out_ref.at[pl.ds((n0 + c * NCHUNK) * ROW, NCHUNK * ROW)])
            return carry

        lax.fori_loop(0, CHUNKS, chunk_body, 0)


def kernel(xyz, edges, tables):
    edges = edges.astype(jnp.int32)
    xyz_p = jnp.pad(xyz, ((0, NPAD - N_NODES), (0, 0))).reshape(-1)
    edges_p = jnp.pad(edges, ((0, NPAD - N_NODES), (0, 0))).reshape(-1)
    tab_flat = tables.reshape(-1)

    mesh = plsc.VectorSubcoreMesh(core_axis_name="c", subcore_axis_name="s")
    call = functools.partial(
        pl.kernel,
        out_type=[jax.ShapeDtypeStruct((NPAD * ROW,), jnp.float32)] * NENC,
        mesh=mesh,
        compiler_params=pltpu.CompilerParams(needs_layout_passes=False),
        scratch_types=[
            pltpu.VMEM((NPAD * 3,), jnp.float32),
            pltpu.VMEM((CPT * K_NBR,), jnp.int32),
            pltpu.VMEM((CPT * K_NBR,), jnp.float32),
            pltpu.VMEM((CPT * K_NBR,), jnp.float32),
            pltpu.VMEM((CPT * K_NBR,), jnp.float32),
            pltpu.VMEM((TWORDS,), jnp.float32),
            pltpu.VMEM((NCHUNK * ROW,), jnp.float32),
        ],
    )(_body)
    encs = call(xyz_p, edges_p, tab_flat)
    shape = (NPAD, K_NBR, LEVELS * FEAT)
    outs = []
    for i in range(3):
        pair = jnp.stack([encs[2 * i].reshape(shape), encs[2 * i + 1].reshape(shape)],
                         axis=1)
        outs.append(pair[:N_NODES])
    return tuple(outs)
